# baseline (speedup 1.0000x reference)
import dataclasses
import functools

import jax
import jax.numpy as jnp
from jax import lax
from jax.experimental import pallas as pl
from jax.experimental.pallas import tpu as pltpu
from jax.experimental.pallas import tpu_sc as plsc

N = 10000
D = 128
E = 160000
NUM_REL = 2
EPS = 1e-3

NC = 2
NS = 16
LANES = 16

CHUNK = 32
E_PAD = 163840
TILE_EDGES = E_PAD // (NC * NS)
NCHUNKS = TILE_EDGES // CHUNK
SEG = TILE_EDGES // 2
SEG_CHUNKS = SEG // CHUNK
ROWS_PER_SUBCORE = 624
ROWS_TAIL = N - ROWS_PER_SUBCORE * NS


_BLK = 1000


def _stats_body(x_ref, o_ref):
    i = pl.program_id(0)

    @pl.when(i == 0)
    def _():
        o_ref[...] = jnp.zeros_like(o_ref)

    xb = x_ref[...]
    o_ref[0:1, :] += jnp.sum(xb, axis=0, keepdims=True)
    o_ref[1:2, :] += jnp.sum(xb * xb, axis=0, keepdims=True)

    @pl.when(i == pl.num_programs(0) - 1)
    def _():
        mean = o_ref[0:1, :] / N
        var = o_ref[1:2, :] / N - mean * mean
        o_ref[0:1, :] = mean
        o_ref[1:2, :] = lax.rsqrt(var + EPS)


def _mm_body(x_ref, s_ref, w_ref, y0_ref, y1_ref):
    mean = s_ref[0:1, :]
    rstd = s_ref[1:2, :]
    xn = (x_ref[...] - mean) * rstd
    dn = (((1,), (0,)), ((), ()))
    y0_ref[...] = lax.dot_general(xn, w_ref[0:D, :], dn,
                                  precision=lax.Precision.HIGHEST,
                                  preferred_element_type=jnp.float32)
    y1_ref[...] = lax.dot_general(xn, w_ref[D:2 * D, :], dn,
                                  precision=lax.Precision.HIGHEST,
                                  preferred_element_type=jnp.float32)


def _tc_prepare(x, W):
    stats = pl.pallas_call(
        _stats_body,
        grid=(N // _BLK,),
        in_specs=[pl.BlockSpec((_BLK, D), lambda i: (i, 0))],
        out_specs=pl.BlockSpec((8, D), lambda i: (0, 0)),
        out_shape=jax.ShapeDtypeStruct((8, D), jnp.float32),
    )(x)
    y0, y1 = pl.pallas_call(
        _mm_body,
        grid=(N // _BLK,),
        in_specs=[
            pl.BlockSpec((_BLK, D), lambda i: (i, 0)),
            pl.BlockSpec((8, D), lambda i: (0, 0)),
            pl.BlockSpec((NUM_REL * D, D), lambda i: (0, 0)),
        ],
        out_specs=[
            pl.BlockSpec((_BLK, D), lambda i: (i, 0)),
            pl.BlockSpec((_BLK, D), lambda i: (i, 0)),
        ],
        out_shape=[
            jax.ShapeDtypeStruct((N, D), jnp.float32),
            jax.ShapeDtypeStruct((N, D), jnp.float32),
        ],
    )(x, stats, W)
    return y0, y1


def _add_body(a_ref, o_ref):
    o_ref[...] = a_ref[0, :, :] + a_ref[1, :, :]


def _tc_combine(partials):
    return pl.pallas_call(
        _add_body,
        grid=(N // _BLK,),
        in_specs=[pl.BlockSpec((NC, _BLK, D), lambda i: (0, i, 0))],
        out_specs=pl.BlockSpec((_BLK, D), lambda i: (i, 0)),
        out_shape=jax.ShapeDtypeStruct((N, D), jnp.float32),
    )(partials)


_MESH = plsc.VectorSubcoreMesh(core_axis_name="c", subcore_axis_name="s",
                               num_cores=NC, num_subcores=NS)

_SC_PARAMS = pltpu.CompilerParams()
if "needs_layout_passes" in pltpu.CompilerParams.__dataclass_fields__:
    _SC_PARAMS = dataclasses.replace(_SC_PARAMS, needs_layout_passes=False)


@functools.partial(
    pl.kernel,
    out_type=jax.ShapeDtypeStruct((NC, N, D), jnp.float32),
    mesh=_MESH,
    compiler_params=_SC_PARAMS,
    scratch_types=[
        pltpu.VMEM((SEG,), jnp.int32),
        pltpu.VMEM((SEG_CHUNKS, CHUNK), jnp.int32),
        pltpu.VMEM((SEG,), jnp.float32),
        pltpu.VMEM((8, CHUNK, D), jnp.float32),
        pltpu.VMEM_SHARED((N, D), jnp.float32),
        [pltpu.SemaphoreType.DMA] * 8,
        [pltpu.SemaphoreType.DMA] * 8,
    ],
)
def _sc_spmm(y0_hbm, src0_hbm, dst0_hbm, val0_hbm,
             y1_hbm, src1_hbm, dst1_hbm, val1_hbm,
             out_hbm, src_v, dst_v, val_v, ring_v, acc,
             sem_g, sem_s):
    c = lax.axis_index("c")
    s = lax.axis_index("s")
    tile = c * NS + s

    zero16 = jnp.zeros((LANES,), jnp.float32)
    zbuf = ring_v.at[0]

    @pl.loop(0, CHUNK)
    def _(r):
        for t in range(D // LANES):
            zbuf[r, pl.ds(t * LANES, LANES)] = zero16

    row0 = s * ROWS_PER_SUBCORE
    for k in range(ROWS_PER_SUBCORE // CHUNK):
        pltpu.sync_copy(zbuf, acc.at[pl.ds(row0 + k * CHUNK, CHUNK)])
    rem = ROWS_PER_SUBCORE % CHUNK
    if rem:
        pltpu.sync_copy(zbuf.at[pl.ds(0, rem)],
                        acc.at[pl.ds(row0 + (ROWS_PER_SUBCORE // CHUNK) * CHUNK,
                                     rem)])

    @pl.when(s == 0)
    def _():
        pltpu.sync_copy(zbuf.at[pl.ds(0, ROWS_TAIL)],
                        acc.at[pl.ds(N - ROWS_TAIL, ROWS_TAIL)])

    plsc.subcore_barrier()

    base = tile * TILE_EDGES
    crow = tile * NCHUNKS
    NBUF = 8

    def _scale(rows_ref, vbase):
        @plsc.parallel_loop(0, CHUNK, unroll=2)
        def _(r):
            vj = plsc.load_gather(
                val_v, [jnp.full((LANES,), vbase + r, jnp.int32)])
            for t in range(D // LANES):
                sl = pl.ds(t * LANES, LANES)
                rows_ref[r, sl] = rows_ref[r, sl] * vj

    for (y_hbm, src_hbm, dst_hbm, vv_hbm) in (
            (y0_hbm, src0_hbm, dst0_hbm, val0_hbm),
            (y1_hbm, src1_hbm, dst1_hbm, val1_hbm)):
        for h in range(2):
            pltpu.sync_copy(src_hbm.at[pl.ds(base + h * SEG, SEG)], src_v)
            pltpu.sync_copy(dst_hbm.at[pl.ds(crow + h * SEG_CHUNKS,
                                             SEG_CHUNKS)], dst_v)
            pltpu.sync_copy(vv_hbm.at[pl.ds(base + h * SEG, SEG)], val_v)

            def _gsrc(i):
                return y_hbm.at[src_v.at[pl.ds(i * CHUNK, CHUNK)]]

            for b in range(NBUF - 1):
                pltpu.async_copy(_gsrc(b), ring_v.at[b], sem_g[b])

            @pl.loop(0, SEG_CHUNKS, step=NBUF)
            def _(i):
                for b in range(NBUF):
                    j = i + b
                    buf = ring_v.at[b]
                    pltpu.make_async_copy(_gsrc(j), buf, sem_g[b]).wait()
                    _scale(buf, j * CHUNK)
                    pltpu.async_copy(buf, acc.at[dst_v.at[j]], sem_s[b],
                                     add=True)
                    nb = (b + NBUF - 1) % NBUF
                    jn = j + NBUF - 1
                    jp = jn - NBUF

                    @pl.when((jp >= 0) & (jn < SEG_CHUNKS))
                    def _():
                        jps = jnp.maximum(jp, 0)
                        jns = jnp.minimum(jn, SEG_CHUNKS - 1)
                        pltpu.make_async_copy(
                            ring_v.at[nb], acc.at[dst_v.at[jps]],
                            sem_s[nb]).wait()
                        pltpu.async_copy(_gsrc(jns), ring_v.at[nb],
                                         sem_g[nb])

                    @pl.when((jp < 0) & (jn < SEG_CHUNKS))
                    def _():
                        jns = jnp.minimum(jn, SEG_CHUNKS - 1)
                        pltpu.async_copy(_gsrc(jns), ring_v.at[nb],
                                         sem_g[nb])

            for b in range(NBUF):
                j = SEG_CHUNKS - NBUF + b
                pltpu.make_async_copy(ring_v.at[b], acc.at[dst_v.at[j]],
                                      sem_s[b]).wait()

    plsc.subcore_barrier()

    pltpu.sync_copy(acc.at[pl.ds(row0, ROWS_PER_SUBCORE)],
                    out_hbm.at[c].at[pl.ds(row0, ROWS_PER_SUBCORE)])

    @pl.when(s == 0)
    def _():
        pltpu.sync_copy(acc.at[pl.ds(N - ROWS_TAIL, ROWS_TAIL)],
                        out_hbm.at[c].at[pl.ds(N - ROWS_TAIL, ROWS_TAIL)])


def _pad_edges(edge_index, edge_vals):
    pad = E_PAD - E
    src = jnp.concatenate([edge_index[1], jnp.zeros((pad,), jnp.int32)])
    dst = jnp.concatenate([edge_index[0], jnp.zeros((pad,), jnp.int32)])
    dst = dst.reshape(E_PAD // CHUNK, CHUNK)
    val = jnp.concatenate([edge_vals, jnp.zeros((pad,), jnp.float32)])
    return src, dst, val


def kernel(x, edge_index_r0, edge_vals_r0, edge_index_r1, edge_vals_r1, W):
    y0, y1 = _tc_prepare(x, W)
    src0, dst0, val0 = _pad_edges(edge_index_r0, edge_vals_r0)
    src1, dst1, val1 = _pad_edges(edge_index_r1, edge_vals_r1)
    partials = _sc_spmm(y0, src0, dst0, val0, y1, src1, dst1, val1)
    return _tc_combine(partials)


# speedup vs baseline: 1.1315x; 1.1315x over previous
import dataclasses
import functools

import jax
import jax.numpy as jnp
from jax import lax
from jax.experimental import pallas as pl
from jax.experimental.pallas import tpu as pltpu
from jax.experimental.pallas import tpu_sc as plsc

N = 10000
D = 128
E = 160000
NUM_REL = 2
EPS = 1e-3

NC = 2
NS = 16
LANES = 16

CHUNK = 64
E_PAD = 163840
TILE_EDGES = E_PAD // (NC * NS)
NCHUNKS = TILE_EDGES // CHUNK
SEG = TILE_EDGES // 2
SEG_CHUNKS = SEG // CHUNK
ROWS_PER_SUBCORE = 624
ROWS_TAIL = N - ROWS_PER_SUBCORE * NS


_BLK = 1000


def _stats_body(x_ref, o_ref):
    i = pl.program_id(0)

    @pl.when(i == 0)
    def _():
        o_ref[...] = jnp.zeros_like(o_ref)

    xb = x_ref[...]
    o_ref[0:1, :] += jnp.sum(xb, axis=0, keepdims=True)
    o_ref[1:2, :] += jnp.sum(xb * xb, axis=0, keepdims=True)

    @pl.when(i == pl.num_programs(0) - 1)
    def _():
        mean = o_ref[0:1, :] / N
        var = o_ref[1:2, :] / N - mean * mean
        o_ref[0:1, :] = mean
        o_ref[1:2, :] = lax.rsqrt(var + EPS)


def _mm_body(x_ref, s_ref, w_ref, y0_ref, y1_ref):
    mean = s_ref[0:1, :]
    rstd = s_ref[1:2, :]
    xn = (x_ref[...] - mean) * rstd
    dn = (((1,), (0,)), ((), ()))
    y0_ref[...] = lax.dot_general(xn, w_ref[0:D, :], dn,
                                  precision=lax.Precision.HIGHEST,
                                  preferred_element_type=jnp.float32)
    y1_ref[...] = lax.dot_general(xn, w_ref[D:2 * D, :], dn,
                                  precision=lax.Precision.HIGHEST,
                                  preferred_element_type=jnp.float32)


def _tc_prepare(x, W):
    stats = pl.pallas_call(
        _stats_body,
        grid=(N // _BLK,),
        in_specs=[pl.BlockSpec((_BLK, D), lambda i: (i, 0))],
        out_specs=pl.BlockSpec((8, D), lambda i: (0, 0)),
        out_shape=jax.ShapeDtypeStruct((8, D), jnp.float32),
    )(x)
    y0, y1 = pl.pallas_call(
        _mm_body,
        grid=(N // _BLK,),
        in_specs=[
            pl.BlockSpec((_BLK, D), lambda i: (i, 0)),
            pl.BlockSpec((8, D), lambda i: (0, 0)),
            pl.BlockSpec((NUM_REL * D, D), lambda i: (0, 0)),
        ],
        out_specs=[
            pl.BlockSpec((_BLK, D), lambda i: (i, 0)),
            pl.BlockSpec((_BLK, D), lambda i: (i, 0)),
        ],
        out_shape=[
            jax.ShapeDtypeStruct((N, D), jnp.float32),
            jax.ShapeDtypeStruct((N, D), jnp.float32),
        ],
    )(x, stats, W)
    return y0, y1


def _add_body(a_ref, o_ref):
    o_ref[...] = a_ref[0, :, :] + a_ref[1, :, :]


def _tc_combine(partials):
    return pl.pallas_call(
        _add_body,
        grid=(N // _BLK,),
        in_specs=[pl.BlockSpec((NC, _BLK, D), lambda i: (0, i, 0))],
        out_specs=pl.BlockSpec((_BLK, D), lambda i: (i, 0)),
        out_shape=jax.ShapeDtypeStruct((N, D), jnp.float32),
    )(partials)


_MESH = plsc.VectorSubcoreMesh(core_axis_name="c", subcore_axis_name="s",
                               num_cores=NC, num_subcores=NS)

_SC_PARAMS = pltpu.CompilerParams()
if "needs_layout_passes" in pltpu.CompilerParams.__dataclass_fields__:
    _SC_PARAMS = dataclasses.replace(_SC_PARAMS, needs_layout_passes=False)


@functools.partial(
    pl.kernel,
    out_type=jax.ShapeDtypeStruct((NC, N, D), jnp.float32),
    mesh=_MESH,
    compiler_params=_SC_PARAMS,
    scratch_types=[
        pltpu.VMEM((SEG,), jnp.int32),
        pltpu.VMEM((SEG_CHUNKS, CHUNK), jnp.int32),
        pltpu.VMEM((SEG,), jnp.float32),
        pltpu.VMEM((4, CHUNK, D), jnp.float32),
        pltpu.VMEM_SHARED((N, D), jnp.float32),
        [pltpu.SemaphoreType.DMA] * 4,
        [pltpu.SemaphoreType.DMA] * 4,
    ],
)
def _sc_spmm(y0_hbm, src0_hbm, dst0_hbm, val0_hbm,
             y1_hbm, src1_hbm, dst1_hbm, val1_hbm,
             out_hbm, src_v, dst_v, val_v, ring_v, acc,
             sem_g, sem_s):
    c = lax.axis_index("c")
    s = lax.axis_index("s")
    tile = c * NS + s

    zero16 = jnp.zeros((LANES,), jnp.float32)
    zbuf = ring_v.at[0]

    @pl.loop(0, CHUNK)
    def _(r):
        for t in range(D // LANES):
            zbuf[r, pl.ds(t * LANES, LANES)] = zero16

    row0 = s * ROWS_PER_SUBCORE
    for k in range(ROWS_PER_SUBCORE // CHUNK):
        pltpu.sync_copy(zbuf, acc.at[pl.ds(row0 + k * CHUNK, CHUNK)])
    rem = ROWS_PER_SUBCORE % CHUNK
    if rem:
        pltpu.sync_copy(zbuf.at[pl.ds(0, rem)],
                        acc.at[pl.ds(row0 + (ROWS_PER_SUBCORE // CHUNK) * CHUNK,
                                     rem)])

    @pl.when(s == 0)
    def _():
        pltpu.sync_copy(zbuf.at[pl.ds(0, ROWS_TAIL)],
                        acc.at[pl.ds(N - ROWS_TAIL, ROWS_TAIL)])

    plsc.subcore_barrier()

    base = tile * TILE_EDGES
    crow = tile * NCHUNKS
    NBUF = 4

    def _scale(rows_ref, vbase):
        @plsc.parallel_loop(0, CHUNK, unroll=2)
        def _(r):
            vj = plsc.load_gather(
                val_v, [jnp.full((LANES,), vbase + r, jnp.int32)])
            for t in range(D // LANES):
                sl = pl.ds(t * LANES, LANES)
                rows_ref[r, sl] = rows_ref[r, sl] * vj

    for (y_hbm, src_hbm, dst_hbm, vv_hbm) in (
            (y0_hbm, src0_hbm, dst0_hbm, val0_hbm),
            (y1_hbm, src1_hbm, dst1_hbm, val1_hbm)):
        for h in range(2):
            pltpu.sync_copy(src_hbm.at[pl.ds(base + h * SEG, SEG)], src_v)
            pltpu.sync_copy(dst_hbm.at[pl.ds(crow + h * SEG_CHUNKS,
                                             SEG_CHUNKS)], dst_v)
            pltpu.sync_copy(vv_hbm.at[pl.ds(base + h * SEG, SEG)], val_v)

            def _gsrc(i):
                return y_hbm.at[src_v.at[pl.ds(i * CHUNK, CHUNK)]]

            for b in range(NBUF - 1):
                pltpu.async_copy(_gsrc(b), ring_v.at[b], sem_g[b])

            @pl.loop(0, SEG_CHUNKS, step=NBUF)
            def _(i):
                for b in range(NBUF):
                    j = i + b
                    buf = ring_v.at[b]
                    pltpu.make_async_copy(_gsrc(j), buf, sem_g[b]).wait()
                    _scale(buf, j * CHUNK)
                    pltpu.async_copy(buf, acc.at[dst_v.at[j]], sem_s[b],
                                     add=True)
                    nb = (b + NBUF - 1) % NBUF
                    jn = j + NBUF - 1
                    jp = jn - NBUF

                    @pl.when((jp >= 0) & (jn < SEG_CHUNKS))
                    def _():
                        jps = jnp.maximum(jp, 0)
                        jns = jnp.minimum(jn, SEG_CHUNKS - 1)
                        pltpu.make_async_copy(
                            ring_v.at[nb], acc.at[dst_v.at[jps]],
                            sem_s[nb]).wait()
                        pltpu.async_copy(_gsrc(jns), ring_v.at[nb],
                                         sem_g[nb])

                    @pl.when((jp < 0) & (jn < SEG_CHUNKS))
                    def _():
                        jns = jnp.minimum(jn, SEG_CHUNKS - 1)
                        pltpu.async_copy(_gsrc(jns), ring_v.at[nb],
                                         sem_g[nb])

            for b in range(NBUF):
                j = SEG_CHUNKS - NBUF + b
                pltpu.make_async_copy(ring_v.at[b], acc.at[dst_v.at[j]],
                                      sem_s[b]).wait()

    plsc.subcore_barrier()

    pltpu.sync_copy(acc.at[pl.ds(row0, ROWS_PER_SUBCORE)],
                    out_hbm.at[c].at[pl.ds(row0, ROWS_PER_SUBCORE)])

    @pl.when(s == 0)
    def _():
        pltpu.sync_copy(acc.at[pl.ds(N - ROWS_TAIL, ROWS_TAIL)],
                        out_hbm.at[c].at[pl.ds(N - ROWS_TAIL, ROWS_TAIL)])


def _pad_edges(edge_index, edge_vals):
    pad = E_PAD - E
    src = jnp.concatenate([edge_index[1], jnp.zeros((pad,), jnp.int32)])
    dst = jnp.concatenate([edge_index[0], jnp.zeros((pad,), jnp.int32)])
    dst = dst.reshape(E_PAD // CHUNK, CHUNK)
    val = jnp.concatenate([edge_vals, jnp.zeros((pad,), jnp.float32)])
    return src, dst, val


def kernel(x, edge_index_r0, edge_vals_r0, edge_index_r1, edge_vals_r1, W):
    y0, y1 = _tc_prepare(x, W)
    src0, dst0, val0 = _pad_edges(edge_index_r0, edge_vals_r0)
    src1, dst1, val1 = _pad_edges(edge_index_r1, edge_vals_r1)
    partials = _sc_spmm(y0, src0, dst0, val0, y1, src1, dst1, val1)
    return _tc_combine(partials)
